# Initial kernel scaffold; baseline (speedup 1.0000x reference)
import jax
import jax.numpy as jnp
from jax.experimental import pallas as pl


def kernel(x, W, b):
    raise NotImplementedError("write your pallas kernel here")


# speedup vs baseline: 1.0910x; 1.0910x over previous
import functools

import jax
import jax.numpy as jnp
from jax.experimental import pallas as pl
from jax.experimental.pallas import tpu as pltpu

_E = 64
_K = 8
_TB = 512


def _router_block(x_ref, w_ref, b_ref, logits_ref, gates_ref, idx_ref):
    l = jnp.dot(x_ref[...], w_ref[...], preferred_element_type=jnp.float32)
    l = l + b_ref[...]
    logits_ref[...] = l
    m = jnp.max(l, axis=1, keepdims=True)
    e = jnp.exp(l - m)
    s = jnp.sum(e, axis=1, keepdims=True)
    p = e / s
    iota = jax.lax.broadcasted_iota(jnp.int32, (_TB, _E), 1)
    work = p
    gates = []
    idxs = []
    for _ in range(_K):
        mx = jnp.max(work, axis=1, keepdims=True)
        am = jnp.min(jnp.where(work == mx, iota, _E), axis=1, keepdims=True)
        gates.append(mx)
        idxs.append(am)
        work = jnp.where(iota == am, -1.0, work)
    g = jnp.concatenate(gates, axis=1)
    i = jnp.concatenate(idxs, axis=1)
    gates_ref[...] = g / (jnp.sum(g, axis=1, keepdims=True) + 1e-9)
    idx_ref[...] = i


@functools.partial(jax.jit, static_argnames=("interpret",))
def _router(x2d, W, b2d, interpret=False):
    T = x2d.shape[0]
    grid = (T // _TB,)
    return pl.pallas_call(
        _router_block,
        grid=grid,
        in_specs=[
            pl.BlockSpec((_TB, x2d.shape[1]), lambda i: (i, 0)),
            pl.BlockSpec((x2d.shape[1], _E), lambda i: (0, 0)),
            pl.BlockSpec((1, _E), lambda i: (0, 0)),
        ],
        out_specs=[
            pl.BlockSpec((_TB, _E), lambda i: (i, 0)),
            pl.BlockSpec((_TB, _K), lambda i: (i, 0)),
            pl.BlockSpec((_TB, _K), lambda i: (i, 0)),
        ],
        out_shape=[
            jax.ShapeDtypeStruct((T, _E), jnp.float32),
            jax.ShapeDtypeStruct((T, _K), jnp.float32),
            jax.ShapeDtypeStruct((T, _K), jnp.int32),
        ],
        compiler_params=pltpu.CompilerParams(
            dimension_semantics=("parallel",),
        ),
        interpret=interpret,
    )(x2d, W, b2d)


def kernel(x, W, b):
    B, S, D = x.shape
    x2d = x.reshape(B * S, D)
    logits, gates, idx = _router(x2d, W, b.reshape(1, _E))
    return (
        gates.reshape(B, S, _K),
        idx.reshape(B, S, _K),
        logits.reshape(B, S, _E),
    )


# speedup vs baseline: 1.1391x; 1.0441x over previous
import functools

import jax
import jax.numpy as jnp
from jax import lax
from jax.experimental import pallas as pl
from jax.experimental.pallas import tpu as pltpu
from jax.experimental.pallas import tpu_sc as plsc

_E = 64
_K = 8
_TB = 512
_NC = 2
_NS = 16
_NW = _NC * _NS
_L = 16


def _mm_block(x_ref, w_ref, b_ref, bt_ref, logits_ref, probs_t_ref):
    x = x_ref[...]
    w = w_ref[...]
    l = jnp.dot(x, w, preferred_element_type=jnp.float32) + b_ref[...]
    logits_ref[...] = l
    lt = jax.lax.dot_general(
        w, x, (((0,), (1,)), ((), ())),
        preferred_element_type=jnp.float32,
    ) + bt_ref[...]
    m = jnp.max(lt, axis=0, keepdims=True)
    e = jnp.exp(lt - m)
    probs_t_ref[...] = e / jnp.sum(e, axis=0, keepdims=True)


def _matmul_probs(x2d, W, b2d, bt2d):
    T, D = x2d.shape
    return pl.pallas_call(
        _mm_block,
        grid=(T // _TB,),
        in_specs=[
            pl.BlockSpec((_TB, D), lambda i: (i, 0)),
            pl.BlockSpec((D, _E), lambda i: (0, 0)),
            pl.BlockSpec((1, _E), lambda i: (0, 0)),
            pl.BlockSpec((_E, 1), lambda i: (0, 0)),
        ],
        out_specs=[
            pl.BlockSpec((_TB, _E), lambda i: (i, 0)),
            pl.BlockSpec((_E, _TB), lambda i: (0, i)),
        ],
        out_shape=[
            jax.ShapeDtypeStruct((T, _E), jnp.float32),
            jax.ShapeDtypeStruct((_E, T), jnp.float32),
        ],
        compiler_params=pltpu.CompilerParams(
            dimension_semantics=("parallel",),
        ),
    )(x2d, W, b2d, bt2d)


def _sc_route_body(tpw, probs_t_hbm, gates_t_hbm, idx_t_hbm, p_v, g_v, i_v):
    wid = lax.axis_index("s") * _NC + lax.axis_index("c")
    base = wid * tpw
    pltpu.sync_copy(probs_t_hbm.at[:, pl.ds(base, tpw)], p_v)

    def group(g, carry):
        off = g * _L
        svals = [jnp.full((_L,), -1.0, jnp.float32) for _ in range(_K)]
        sidx = [jnp.zeros((_L,), jnp.int32) for _ in range(_K)]
        for e in range(_E):
            cv = p_v[e, pl.ds(off, _L)]
            ci = jnp.full((_L,), e, jnp.int32)
            for i in range(_K):
                m = cv > svals[i]
                sv, si = svals[i], sidx[i]
                svals[i] = jnp.where(m, cv, sv)
                sidx[i] = jnp.where(m, ci, si)
                cv = jnp.where(m, sv, cv)
                ci = jnp.where(m, si, ci)
        tot = svals[0]
        for i in range(1, _K):
            tot = tot + svals[i]
        tot = tot + 1e-9
        for i in range(_K):
            g_v[i, pl.ds(off, _L)] = svals[i] / tot
            i_v[i, pl.ds(off, _L)] = sidx[i]
        return carry

    lax.fori_loop(0, tpw // _L, group, 0)
    pltpu.sync_copy(g_v, gates_t_hbm.at[:, pl.ds(base, tpw)])
    pltpu.sync_copy(i_v, idx_t_hbm.at[:, pl.ds(base, tpw)])


def _sc_route(probs_t):
    T = probs_t.shape[1]
    tpw = T // _NW
    mesh = plsc.VectorSubcoreMesh(core_axis_name="c", subcore_axis_name="s")
    f = functools.partial(
        pl.kernel,
        mesh=mesh,
        out_type=[
            jax.ShapeDtypeStruct((_K, T), jnp.float32),
            jax.ShapeDtypeStruct((_K, T), jnp.int32),
        ],
        scratch_types=[
            pltpu.VMEM((_E, tpw), jnp.float32),
            pltpu.VMEM((_K, tpw), jnp.float32),
            pltpu.VMEM((_K, tpw), jnp.int32),
        ],
    )(functools.partial(_sc_route_body, tpw))
    return f(probs_t)


@jax.jit
def _router(x2d, W, b2d, bt2d):
    logits, probs_t = _matmul_probs(x2d, W, b2d, bt2d)
    gates_t, idx_t = _sc_route(probs_t)
    return logits, gates_t.T, idx_t.T


def kernel(x, W, b):
    B, S, D = x.shape
    x2d = x.reshape(B * S, D)
    logits, gates, idx = _router(x2d, W, b.reshape(1, _E), b.reshape(_E, 1))
    return (
        gates.reshape(B, S, _K),
        idx.reshape(B, S, _K),
        logits.reshape(B, S, _E),
    )


# speedup vs baseline: 1.2352x; 1.0844x over previous
import functools

import jax
import jax.numpy as jnp
from jax import lax
from jax.experimental import pallas as pl
from jax.experimental.pallas import tpu as pltpu
from jax.experimental.pallas import tpu_sc as plsc

_E = 64
_K = 8
_TB = 1024
_NC = 2
_NS = 16
_NW = _NC * _NS
_L = 16


def _mm_block(x_ref, w_ref, b_ref, bt_ref, logits_ref, probs_t_ref):
    x = x_ref[...]
    w = w_ref[...]
    l = jnp.dot(x, w, preferred_element_type=jnp.float32) + b_ref[...]
    logits_ref[...] = l
    lt = jax.lax.dot_general(
        w, x, (((0,), (1,)), ((), ())),
        preferred_element_type=jnp.float32,
    ) + bt_ref[...]
    m = jnp.max(lt, axis=0, keepdims=True)
    e = jnp.exp(lt - m)
    probs_t_ref[...] = e / jnp.sum(e, axis=0, keepdims=True)


def _matmul_probs(x2d, W, b2d, bt2d):
    T, D = x2d.shape
    return pl.pallas_call(
        _mm_block,
        grid=(T // _TB,),
        in_specs=[
            pl.BlockSpec((_TB, D), lambda i: (i, 0)),
            pl.BlockSpec((D, _E), lambda i: (0, 0)),
            pl.BlockSpec((1, _E), lambda i: (0, 0)),
            pl.BlockSpec((_E, 1), lambda i: (0, 0)),
        ],
        out_specs=[
            pl.BlockSpec((_TB, _E), lambda i: (i, 0)),
            pl.BlockSpec((_E, _TB), lambda i: (0, i)),
        ],
        out_shape=[
            jax.ShapeDtypeStruct((T, _E), jnp.float32),
            jax.ShapeDtypeStruct((_E, T), jnp.float32),
        ],
        compiler_params=pltpu.CompilerParams(
            dimension_semantics=("parallel",),
        ),
    )(x2d, W, b2d, bt2d)


def _sc_route_body(tpw, probs_t_hbm, gates_t_hbm, idx_t_hbm, p_v, g_v, i_v):
    wid = lax.axis_index("s") * _NC + lax.axis_index("c")
    base = wid * tpw
    pltpu.sync_copy(probs_t_hbm.at[:, pl.ds(base, tpw)], p_v)

    def group(g, carry):
        off = g * _L
        svals = [jnp.full((_L,), -1.0, jnp.float32) for _ in range(_K)]
        sidx = [jnp.zeros((_L,), jnp.int32) for _ in range(_K)]
        for e in range(_E):
            cv = p_v[e, pl.ds(off, _L)]
            ci = jnp.full((_L,), e, jnp.int32)
            for i in range(_K):
                m = cv > svals[i]
                sv, si = svals[i], sidx[i]
                svals[i] = jnp.where(m, cv, sv)
                sidx[i] = jnp.where(m, ci, si)
                cv = jnp.where(m, sv, cv)
                ci = jnp.where(m, si, ci)
        tot = svals[0]
        for i in range(1, _K):
            tot = tot + svals[i]
        tot = tot + 1e-9
        for i in range(_K):
            g_v[i, pl.ds(off, _L)] = svals[i] / tot
            i_v[i, pl.ds(off, _L)] = sidx[i]
        return carry

    lax.fori_loop(0, tpw // _L, group, 0)
    pltpu.sync_copy(g_v, gates_t_hbm.at[:, pl.ds(base, tpw)])
    pltpu.sync_copy(i_v, idx_t_hbm.at[:, pl.ds(base, tpw)])


def _sc_route(probs_t):
    T = probs_t.shape[1]
    tpw = T // _NW
    mesh = plsc.VectorSubcoreMesh(core_axis_name="c", subcore_axis_name="s")
    f = functools.partial(
        pl.kernel,
        mesh=mesh,
        out_type=[
            jax.ShapeDtypeStruct((_K, T), jnp.float32),
            jax.ShapeDtypeStruct((_K, T), jnp.int32),
        ],
        scratch_types=[
            pltpu.VMEM((_E, tpw), jnp.float32),
            pltpu.VMEM((_K, tpw), jnp.float32),
            pltpu.VMEM((_K, tpw), jnp.int32),
        ],
    )(functools.partial(_sc_route_body, tpw))
    return f(probs_t)


@jax.jit
def _router(x2d, W, b2d, bt2d):
    logits, probs_t = _matmul_probs(x2d, W, b2d, bt2d)
    gates_t, idx_t = _sc_route(probs_t)
    return logits, gates_t.T, idx_t.T


def kernel(x, W, b):
    B, S, D = x.shape
    x2d = x.reshape(B * S, D)
    logits, gates, idx = _router(x2d, W, b.reshape(1, _E), b.reshape(_E, 1))
    return (
        gates.reshape(B, S, _K),
        idx.reshape(B, S, _K),
        logits.reshape(B, S, _E),
    )
